# baseline (speedup 1.0000x reference)
import functools

import jax
import jax.numpy as jnp
from jax import lax
from jax.experimental import pallas as pl
from jax.experimental.pallas import tpu as pltpu
from jax.experimental.pallas import tpu_sc as plsc

B = 4096
FD = 13
FS = 26
V = 100000
D = 16
H1, H2, H3 = 256, 128, 64
OUT = 10

NC, NS = 2, 16
NW = NC * NS
P2 = FS * D
PPW = P2 // NW
NCHUNK = B // 16


def _sc_gather(tab2_hbm, tab1_hbm, idx_hbm, g2t_hbm, g1t_hbm,
               plane_v, idx_v, vals_v, psem, isem, osem):
    wid = lax.axis_index("s") * NC + lax.axis_index("c")

    def gather_plane(_, carry):
        def inner(c, carry2):
            v = idx_v[pl.ds(c * 16, 16)]
            vals_v[pl.ds(c * 16, 16)] = plsc.load_gather(plane_v, [v])
            return carry2
        lax.fori_loop(0, NCHUNK, inner, carry)

    def do_plane2(p, carry):
        pltpu.async_copy(idx_hbm.at[p // D], idx_v, isem)
        pltpu.async_copy(tab2_hbm.at[p], plane_v, psem)
        pltpu.make_async_copy(idx_hbm.at[p // D], idx_v, isem).wait()
        pltpu.make_async_copy(tab2_hbm.at[p], plane_v, psem).wait()
        @pl.when(p > wid * PPW)
        def _():
            pltpu.make_async_copy(vals_v, g2t_hbm.at[p - 1], osem).wait()
        gather_plane(p, carry)
        pltpu.async_copy(vals_v, g2t_hbm.at[p], osem)
        return carry

    lax.fori_loop(wid * PPW, (wid + 1) * PPW, do_plane2, 0)
    pltpu.make_async_copy(vals_v, g2t_hbm.at[0], osem).wait()

    @pl.when(wid < FS)
    def _():
        pltpu.async_copy(idx_hbm.at[wid], idx_v, isem)
        pltpu.async_copy(tab1_hbm.at[wid, 0], plane_v, psem)
        pltpu.make_async_copy(idx_hbm.at[wid], idx_v, isem).wait()
        pltpu.make_async_copy(tab1_hbm.at[wid, 0], plane_v, psem).wait()
        gather_plane(wid, 0)
        pltpu.sync_copy(vals_v, g1t_hbm.at[wid])


@functools.cache
def _sc_gather_call():
    return functools.partial(
        pl.kernel,
        mesh=plsc.VectorSubcoreMesh(core_axis_name="c", subcore_axis_name="s"),
        compiler_params=pltpu.CompilerParams(needs_layout_passes=False),
        out_type=[
            jax.ShapeDtypeStruct((P2, B), jnp.float32),
            jax.ShapeDtypeStruct((FS, B), jnp.float32),
        ],
        scratch_types=[
            pltpu.VMEM((V,), jnp.float32),
            pltpu.VMEM((B,), jnp.int32),
            pltpu.VMEM((B,), jnp.float32),
            pltpu.SemaphoreType.DMA,
            pltpu.SemaphoreType.DMA,
            pltpu.SemaphoreType.DMA,
        ],
    )(_sc_gather)


def _tc_dense(g2t_ref, g1t_ref, dense_ref, w1a_ref, w1b_ref, b1_ref,
              w2_ref, b2_ref, w3_ref, b3_ref, wf_ref, bf_ref,
              wd_ref, bd_ref, out_ref):
    f32 = jnp.float32
    cW = (((0,), (0,)), ((), ()))
    cD = (((0,), (1,)), ((), ()))
    g2t = g2t_ref[...]
    dense = dense_ref[...]
    r = lax.broadcasted_iota(jnp.int32, (D, P2), 1)
    c = lax.broadcasted_iota(jnp.int32, (D, P2), 0)
    sel = (r % D == c).astype(f32)
    sum_e = jnp.dot(sel, g2t, preferred_element_type=f32)
    se2 = jnp.dot(jnp.ones((8, D), f32), sum_e * sum_e,
                  preferred_element_type=f32)[:1]
    sq = jnp.dot(jnp.ones((8, P2), f32), g2t * g2t,
                 preferred_element_type=f32)[:1]
    fm2 = 0.5 * (se2 - sq)
    fm1 = (jnp.dot(jnp.ones((8, FS), f32), g1t_ref[...],
                   preferred_element_type=f32)[:1]
           + lax.dot_general(wd_ref[...], dense, cD, preferred_element_type=f32)
           + bd_ref[...])
    h = lax.dot_general(w1a_ref[...], g2t, cW, preferred_element_type=f32)
    h += lax.dot_general(w1b_ref[...], dense, cD, preferred_element_type=f32)
    h = jnp.maximum(h + b1_ref[...], 0.0)
    h = jnp.maximum(lax.dot_general(w2_ref[...], h, cW, preferred_element_type=f32)
                    + b2_ref[...], 0.0)
    h = jnp.maximum(lax.dot_general(w3_ref[...], h, cW, preferred_element_type=f32)
                    + b3_ref[...], 0.0)
    dnn = (lax.dot_general(wf_ref[...], h, cW, preferred_element_type=f32)
           + bf_ref[...])
    out_ref[...] = dnn + fm1 + fm2


def _tc_call(g2t, g1t, dense, w1a, w1b, b1t, w2, b2t, w3, b3t,
             wf, bft, wd, bdt):
    BB = 2048
    grid = (B // BB,)
    col_spec = lambda rows: pl.BlockSpec((rows, BB), lambda i: (0, i))
    row_spec = lambda cols: pl.BlockSpec((BB, cols), lambda i: (i, 0))
    full = lambda a: pl.BlockSpec(a.shape, lambda i: (0,) * a.ndim)
    return pl.pallas_call(
        _tc_dense,
        grid=grid,
        in_specs=[
            col_spec(P2), col_spec(FS), row_spec(FD),
            full(w1a), full(w1b), full(b1t), full(w2), full(b2t),
            full(w3), full(b3t), full(wf), full(bft), full(wd), full(bdt),
        ],
        out_specs=col_spec(OUT),
        out_shape=jax.ShapeDtypeStruct((OUT, B), jnp.float32),
    )(g2t, g1t, dense, w1a, w1b, b1t, w2, b2t, w3, b3t, wf, bft, wd, bdt)


def kernel(target_x, emb1, emb2, Wd, bd, W1, b1, W2, b2, W3, b3, Wf, bf):
    dense = target_x[:, :FD]
    idx_t = target_x[:, FD:].astype(jnp.int32).T
    tab2 = emb2.transpose(0, 2, 1).reshape(P2, V)
    tab1 = emb1.transpose(0, 2, 1)
    g2t, g1t = _sc_gather_call()(tab2, tab1, idx_t)
    out_t = _tc_call(
        g2t, g1t, dense,
        W1[:P2], W1[P2:], b1.reshape(H1, 1),
        W2, b2.reshape(H2, 1), W3, b3.reshape(H3, 1),
        Wf, bf.reshape(OUT, 1), Wd, bd.reshape(1, 1),
    )
    return out_t.T


# speedup vs baseline: 1.0088x; 1.0088x over previous
import functools

import jax
import jax.numpy as jnp
from jax import lax
from jax.experimental import pallas as pl
from jax.experimental.pallas import tpu as pltpu
from jax.experimental.pallas import tpu_sc as plsc

B = 4096
FD = 13
FS = 26
V = 100000
D = 16
H1, H2, H3 = 256, 128, 64
OUT = 10

NC, NS = 2, 16
NW = NC * NS
P2 = FS * D
PPW = P2 // NW
NCHUNK = B // 16


def _sc_gather(tab2_hbm, tab1_hbm, idx_hbm, g2t_hbm, g1t_hbm,
               plane_v, idx_v, vals_v, psem, isem, osem):
    wid = lax.axis_index("s") * NC + lax.axis_index("c")

    def gather_plane(_, carry):
        def inner(c, carry2):
            v = idx_v[pl.ds(c * 16, 16)]
            vals_v[pl.ds(c * 16, 16)] = plsc.load_gather(plane_v, [v])
            return carry2
        lax.fori_loop(0, NCHUNK, inner, carry)

    def do_plane2(p, carry):
        pltpu.async_copy(idx_hbm.at[p // D], idx_v, isem)
        pltpu.async_copy(tab2_hbm.at[p], plane_v, psem)
        pltpu.make_async_copy(idx_hbm.at[p // D], idx_v, isem).wait()
        pltpu.make_async_copy(tab2_hbm.at[p], plane_v, psem).wait()
        @pl.when(p > wid * PPW)
        def _():
            pltpu.make_async_copy(vals_v, g2t_hbm.at[p - 1], osem).wait()
        gather_plane(p, carry)
        pltpu.async_copy(vals_v, g2t_hbm.at[p], osem)
        return carry

    lax.fori_loop(wid * PPW, (wid + 1) * PPW, do_plane2, 0)
    pltpu.make_async_copy(vals_v, g2t_hbm.at[0], osem).wait()

    @pl.when(wid < FS)
    def _():
        pltpu.async_copy(idx_hbm.at[wid], idx_v, isem)
        pltpu.async_copy(tab1_hbm.at[wid, 0], plane_v, psem)
        pltpu.make_async_copy(idx_hbm.at[wid], idx_v, isem).wait()
        pltpu.make_async_copy(tab1_hbm.at[wid, 0], plane_v, psem).wait()
        gather_plane(wid, 0)
        pltpu.sync_copy(vals_v, g1t_hbm.at[wid])


@functools.cache
def _sc_gather_call():
    return functools.partial(
        pl.kernel,
        mesh=plsc.VectorSubcoreMesh(core_axis_name="c", subcore_axis_name="s"),
        compiler_params=pltpu.CompilerParams(needs_layout_passes=False),
        out_type=[
            jax.ShapeDtypeStruct((P2, B), jnp.float32),
            jax.ShapeDtypeStruct((FS, B), jnp.float32),
        ],
        scratch_types=[
            pltpu.VMEM((V,), jnp.float32),
            pltpu.VMEM((B,), jnp.int32),
            pltpu.VMEM((B,), jnp.float32),
            pltpu.SemaphoreType.DMA,
            pltpu.SemaphoreType.DMA,
            pltpu.SemaphoreType.DMA,
        ],
    )(_sc_gather)


def _tc_dense(g2t_ref, g1t_ref, denset_ref, w1at_ref, w1bt_ref, b1_ref,
              w2t_ref, b2_ref, w3t_ref, b3_ref, wft_ref, bf_ref,
              wdt_ref, bd_ref, out_ref):
    f32 = jnp.float32
    g2t = g2t_ref[...]
    denset = denset_ref[...]
    r = lax.broadcasted_iota(jnp.int32, (D, P2), 1)
    c = lax.broadcasted_iota(jnp.int32, (D, P2), 0)
    sel = (r % D == c).astype(f32)
    sum_e = jnp.dot(sel, g2t, preferred_element_type=f32)
    se2 = jnp.dot(jnp.ones((8, D), f32), sum_e * sum_e,
                  preferred_element_type=f32)[:1]
    sq = jnp.dot(jnp.ones((8, P2), f32), g2t * g2t,
                 preferred_element_type=f32)[:1]
    fm2 = 0.5 * (se2 - sq)
    fm1 = (jnp.dot(jnp.ones((8, FS), f32), g1t_ref[...],
                   preferred_element_type=f32)[:1]
           + jnp.dot(wdt_ref[...], denset, preferred_element_type=f32)
           + bd_ref[...])
    h = jnp.dot(w1at_ref[...], g2t, preferred_element_type=f32)
    h += jnp.dot(w1bt_ref[...], denset, preferred_element_type=f32)
    h = jnp.maximum(h + b1_ref[...], 0.0)
    h = jnp.maximum(jnp.dot(w2t_ref[...], h, preferred_element_type=f32)
                    + b2_ref[...], 0.0)
    h = jnp.maximum(jnp.dot(w3t_ref[...], h, preferred_element_type=f32)
                    + b3_ref[...], 0.0)
    dnn = jnp.dot(wft_ref[...], h, preferred_element_type=f32) + bf_ref[...]
    out_ref[...] = dnn + fm1 + fm2


def _tc_call(g2t, g1t, denset, w1at, w1bt, b1t, w2t, b2t, w3t, b3t,
             wft, bft, wdt, bdt):
    BB = 2048
    grid = (B // BB,)
    col_spec = lambda rows: pl.BlockSpec((rows, BB), lambda i: (0, i))
    full = lambda a: pl.BlockSpec(a.shape, lambda i: (0,) * a.ndim)
    return pl.pallas_call(
        _tc_dense,
        grid=grid,
        in_specs=[
            col_spec(P2), col_spec(FS), col_spec(FD),
            full(w1at), full(w1bt), full(b1t), full(w2t), full(b2t),
            full(w3t), full(b3t), full(wft), full(bft), full(wdt), full(bdt),
        ],
        out_specs=col_spec(OUT),
        out_shape=jax.ShapeDtypeStruct((OUT, B), jnp.float32),
    )(g2t, g1t, denset, w1at, w1bt, b1t, w2t, b2t, w3t, b3t, wft, bft, wdt, bdt)


def kernel(target_x, emb1, emb2, Wd, bd, W1, b1, W2, b2, W3, b3, Wf, bf):
    denset = target_x[:, :FD].T
    idx_t = target_x[:, FD:].astype(jnp.int32).T
    tab2 = emb2.transpose(0, 2, 1).reshape(P2, V)
    tab1 = emb1.transpose(0, 2, 1)
    g2t, g1t = _sc_gather_call()(tab2, tab1, idx_t)
    out_t = _tc_call(
        g2t, g1t, denset,
        W1[:P2].T, W1[P2:].T, b1.reshape(H1, 1),
        W2.T, b2.reshape(H2, 1), W3.T, b3.reshape(H3, 1),
        Wf.T, bf.reshape(OUT, 1), Wd.T, bd.reshape(1, 1),
    )
    return out_t.T
